# baseline (device time: 41503 ns/iter reference)
import jax
import jax.numpy as jnp
from jax import lax
from jax.experimental import pallas as pl
from jax.experimental.pallas import tpu as pltpu

M = 1024
N = 1024
K = 4096
Q = 512
HB = Q // 2
NC = 4
CC = Q // NC


def kernel(dy, W):
    def body(dy_ref, w_ref, out_ref,
             dyb_ref, wbuf_ref, part_ref, yrecv_ref,
             ldma_sems,
             y_send, y_recv, x_send, x_recv,
             za_send, za_recv, zb_send, zb_recv, xb_send, xb_recv):
        mx = lax.axis_index("x")
        my = lax.axis_index("y")
        mz = lax.axis_index("z")
        r0 = mz * Q
        rz = (1 - mz) * Q
        c0 = mx * Q
        cx = (1 - mx) * Q

        dy_cp = pltpu.make_async_copy(
            dy_ref.at[pl.ds(r0, Q), :], dyb_ref, ldma_sems.at[2])
        dy_cp.start()
        w_cps = [
            pltpu.make_async_copy(
                w_ref.at[pl.ds(c0 + c * CC, CC), :], wbuf_ref.at[c % 2],
                ldma_sems.at[c % 2])
            for c in range(NC)
        ]
        w_cps[0].start()

        barrier = pltpu.get_barrier_semaphore()
        for nbr in ((1 - mx, my, mz), (mx, 1 - my, mz), (mx, my, 1 - mz)):
            pl.semaphore_signal(
                barrier, inc=1, device_id=nbr,
                device_id_type=pl.DeviceIdType.MESH,
            )
        pl.semaphore_wait(barrier, 3)

        def rdma(src, dst, ssem, rsem, dev):
            return pltpu.make_async_remote_copy(
                src_ref=src, dst_ref=dst, send_sem=ssem, recv_sem=rsem,
                device_id=dev, device_id_type=pl.DeviceIdType.MESH)

        y_nbr = (mx, 1 - my, mz)
        x_nbr = (1 - mx, my, mz)
        z_nbr = (mx, my, 1 - mz)

        y_rdmas, x_rdmas, za_rdmas, zb_rdmas, xb_rdmas = [], [], [], [], []
        for c in range(NC):
            cols = pl.ds(c0 + c * CC, CC)
            colsx = pl.ds(cx + c * CC, CC)
            y_rdmas.append(rdma(part_ref.at[c], yrecv_ref.at[c],
                                y_send.at[c], y_recv.at[c], y_nbr))
            x_rdmas.append(rdma(out_ref.at[pl.ds(r0, Q), cols],
                                out_ref.at[pl.ds(r0, Q), cols],
                                x_send.at[c], x_recv.at[c], x_nbr))
            za_rdmas.append(rdma(out_ref.at[pl.ds(r0, Q), cols],
                                 out_ref.at[pl.ds(r0, Q), cols],
                                 za_send.at[c], za_recv.at[c], z_nbr))
            zb_rdmas.append(rdma(out_ref.at[pl.ds(r0, HB), colsx],
                                 out_ref.at[pl.ds(r0, HB), colsx],
                                 zb_send.at[c], zb_recv.at[c], z_nbr))
            xb_rdmas.append(rdma(out_ref.at[pl.ds(rz + HB, HB), cols],
                                 out_ref.at[pl.ds(rz + HB, HB), cols],
                                 xb_send.at[c], xb_recv.at[c], x_nbr))

        dy_cp.wait()
        for c in range(NC):
            w_cps[c].wait()
            if c + 1 < NC:
                w_cps[c + 1].start()
            part_ref[c] = lax.dot_general(
                dyb_ref[...], wbuf_ref[c % 2],
                dimension_numbers=(((1,), (1,)), ((), ())),
                preferred_element_type=jnp.float32,
            )
            y_rdmas[c].start()

        for c in range(NC):
            y_rdmas[c].wait()
            out_ref[pl.ds(r0, Q), pl.ds(c0 + c * CC, CC)] = (
                part_ref[c] + yrecv_ref[c])
            x_rdmas[c].start()
            za_rdmas[c].start()

        for c in range(NC):
            x_rdmas[c].wait()
            zb_rdmas[c].start()

        for c in range(NC):
            za_rdmas[c].wait()
            xb_rdmas[c].start()

        for c in range(NC):
            zb_rdmas[c].wait()
            xb_rdmas[c].wait()

    return pl.pallas_call(
        body,
        out_shape=jax.ShapeDtypeStruct((M, N), jnp.float32),
        in_specs=[
            pl.BlockSpec(memory_space=pl.ANY),
            pl.BlockSpec(memory_space=pl.ANY),
        ],
        out_specs=pl.BlockSpec(memory_space=pltpu.VMEM),
        scratch_shapes=[
            pltpu.VMEM((Q, K), jnp.float32),
            pltpu.VMEM((2, CC, K), jnp.float32),
            pltpu.VMEM((NC, Q, CC), jnp.float32),
            pltpu.VMEM((NC, Q, CC), jnp.float32),
            pltpu.SemaphoreType.DMA((3,)),
            pltpu.SemaphoreType.DMA((NC,)),
            pltpu.SemaphoreType.DMA((NC,)),
            pltpu.SemaphoreType.DMA((NC,)),
            pltpu.SemaphoreType.DMA((NC,)),
            pltpu.SemaphoreType.DMA((NC,)),
            pltpu.SemaphoreType.DMA((NC,)),
            pltpu.SemaphoreType.DMA((NC,)),
            pltpu.SemaphoreType.DMA((NC,)),
            pltpu.SemaphoreType.DMA((NC,)),
            pltpu.SemaphoreType.DMA((NC,)),
        ],
        compiler_params=pltpu.CompilerParams(collective_id=0),
    )(dy, W)
